# baseline (speedup 1.0000x reference)
import functools

import jax
import jax.numpy as jnp
from jax import lax
from jax.experimental import pallas as pl
from jax.experimental.pallas import tpu as pltpu
from jax.experimental.pallas import tpu_sc as plsc

N = 10000
E = 320000
D = 128
H = 128
C = 47
CP = 128

NC = 2
NS = 16
NT = NC * NS
EPT = E // NT
K = 80
NCHUNK = EPT // K
NP = 10240
RPT = NP // NS
RB = 128
NB = RPT // RB


def _mm_body(h_ref, w_ref, o_ref):
    o_ref[...] = jnp.dot(h_ref[...], w_ref[...],
                         precision=lax.Precision.HIGHEST,
                         preferred_element_type=jnp.float32)


def _matmul(h, w):
    n, d = h.shape
    m = w.shape[1]
    bn = 512
    return pl.pallas_call(
        _mm_body,
        grid=(pl.cdiv(n, bn),),
        in_specs=[pl.BlockSpec((bn, d), lambda i: (i, 0)),
                  pl.BlockSpec((d, m), lambda i: (0, 0))],
        out_specs=pl.BlockSpec((bn, m), lambda i: (i, 0)),
        out_shape=jax.ShapeDtypeStruct((n, m), jnp.float32),
    )(h, w)


def _combine_body(relu, h_ref, w_ref, b_ref, a0_ref, a1_ref, dp_ref, o_ref):
    deg = jnp.sum(dp_ref[...], axis=0)[:, None]
    inv = 1.0 / jnp.maximum(deg, 1.0)
    o = (jnp.dot(h_ref[...], w_ref[...],
                 precision=lax.Precision.HIGHEST,
                 preferred_element_type=jnp.float32)
         + b_ref[...] + (a0_ref[...] + a1_ref[...]) * inv)
    if relu:
        o = jnp.maximum(o, 0.0)
    o_ref[...] = o


def _combine(h, w, b, a0, a1, dp, relu):
    n, d = h.shape
    m = w.shape[1]
    bn = 512
    return pl.pallas_call(
        functools.partial(_combine_body, relu),
        grid=(pl.cdiv(n, bn),),
        in_specs=[pl.BlockSpec((bn, d), lambda i: (i, 0)),
                  pl.BlockSpec((d, m), lambda i: (0, 0)),
                  pl.BlockSpec((1, m), lambda i: (0, 0)),
                  pl.BlockSpec((bn, m), lambda i: (i, 0)),
                  pl.BlockSpec((bn, m), lambda i: (i, 0)),
                  pl.BlockSpec((NT, bn), lambda i: (0, i))],
        out_specs=pl.BlockSpec((bn, m), lambda i: (i, 0)),
        out_shape=jax.ShapeDtypeStruct((n, m), jnp.float32),
    )(h, w, b, a0, a1, dp)


def _make_sc_agg(w, with_deg):
    mesh = plsc.VectorSubcoreMesh(core_axis_name="c", subcore_axis_name="s")
    out_type = [jax.ShapeDtypeStruct((NC, NP, w), jnp.float32)]
    scratch = [
        pltpu.VMEM_SHARED((NP, w), jnp.float32),
        pltpu.VMEM((2, K), jnp.int32),
        pltpu.VMEM((K, w), jnp.float32),
        pltpu.VMEM((RB, w), jnp.float32),
        pltpu.VMEM((RB,), jnp.int32),
        pltpu.SemaphoreType.DMA,
    ]
    if with_deg:
        out_type.append(jax.ShapeDtypeStruct((NT, NP), jnp.float32))
        scratch.append(pltpu.VMEM((NP,), jnp.float32))

    @functools.partial(
        pl.kernel, out_type=out_type, mesh=mesh, scratch_types=scratch,
        compiler_params=pltpu.CompilerParams(needs_layout_passes=False))
    def sc_agg(*refs):
        if with_deg:
            (t_hbm, sd_hbm, z_hbm, zdeg_hbm,
             out_acc, out_deg,
             acc_sh, sd_v, rows_v, bounce_v, ix_v, sem, deg_v) = refs
        else:
            (t_hbm, sd_hbm, z_hbm,
             out_acc,
             acc_sh, sd_v, rows_v, bounce_v, ix_v, sem) = refs
        cid = lax.axis_index("c")
        sid = lax.axis_index("s")
        wid = sid * NC + cid
        row0 = sid * RPT
        iota = lax.iota(jnp.int32, 16)
        pltpu.sync_copy(z_hbm, bounce_v)
        if with_deg:
            pltpu.sync_copy(zdeg_hbm, deg_v)

        def fill_ix(r):
            for ii in range(RB // 16):
                ix_v[pl.ds(ii * 16, 16)] = iota + (r + ii * 16)

        def zero_chunk(i, carry):
            r = row0 + i * RB
            fill_ix(r)
            pltpu.sync_copy(bounce_v, acc_sh.at[ix_v])
            return carry

        lax.fori_loop(0, NB, zero_chunk, 0)
        plsc.subcore_barrier()
        cbase = wid * NCHUNK
        ones_lane = jnp.ones((16,), jnp.float32)

        def chunk(j, carry):
            pltpu.sync_copy(sd_hbm.at[cbase + j], sd_v)
            pltpu.async_copy(t_hbm.at[sd_v.at[0]], rows_v, sem).wait()
            pltpu.sync_copy(rows_v, acc_sh.at[sd_v.at[1]], add=True)
            if with_deg:
                for ii in range(K // 16):
                    idx = sd_v[1, pl.ds(ii * 16, 16)]
                    plsc.addupdate_scatter(deg_v, [idx], ones_lane)
            return carry

        lax.fori_loop(0, NCHUNK, chunk, 0)
        plsc.subcore_barrier()

        def read_chunk(i, carry):
            r = row0 + i * RB
            fill_ix(r)
            pltpu.async_copy(acc_sh.at[ix_v], bounce_v, sem).wait()
            pltpu.sync_copy(bounce_v,
                            out_acc.at[cid, pl.ds(pl.multiple_of(r, 8), RB)])
            return carry

        lax.fori_loop(0, NB, read_chunk, 0)
        if with_deg:
            pltpu.sync_copy(deg_v, out_deg.at[wid])

    return sc_agg


_sc_agg_deg = _make_sc_agg(H, True)
_sc_agg_h = _make_sc_agg(H, False)


def kernel(x, edge_index, W_self0, W_neigh0, b0, W_self1, W_neigh1, b1,
           W_self2, W_neigh2, b2):
    sd = jnp.stack([edge_index[0].reshape(NT * NCHUNK, K),
                    edge_index[1].reshape(NT * NCHUNK, K)], axis=1)
    zH = jnp.zeros((RB, H), jnp.float32)
    zC = jnp.zeros((RB, CP), jnp.float32)
    zdeg = jnp.zeros((NP,), jnp.float32)

    t0 = _matmul(x, W_neigh0)
    acc0, degp = _sc_agg_deg(t0, sd, zH, zdeg)
    dp = degp[:, :N]
    h1 = _combine(x, W_self0, b0.reshape(1, H), acc0[0, :N], acc0[1, :N],
                  dp, relu=True)

    t1 = _matmul(h1, W_neigh1)
    acc1 = _sc_agg_h(t1, sd, zH)[0]
    h2 = _combine(h1, W_self1, b1.reshape(1, H), acc1[0, :N], acc1[1, :N],
                  dp, relu=True)

    Wn2 = jnp.pad(W_neigh2, ((0, 0), (0, CP - C)))
    Ws2 = jnp.pad(W_self2, ((0, 0), (0, CP - C)))
    b2p = jnp.pad(b2, (0, CP - C)).reshape(1, CP)
    t2 = _matmul(h2, Wn2)
    acc2 = _sc_agg_h(t2, sd, zC)[0]
    out = _combine(h2, Ws2, b2p, acc2[0, :N], acc2[1, :N], dp, relu=False)
    return out[:, :C]


# speedup vs baseline: 1.0463x; 1.0463x over previous
import functools

import jax
import jax.numpy as jnp
from jax import lax
from jax.experimental import pallas as pl
from jax.experimental.pallas import tpu as pltpu
from jax.experimental.pallas import tpu_sc as plsc

N = 10000
E = 320000
D = 128
H = 128
C = 47
CP = 128

NC = 2
NS = 16
NT = NC * NS
EPT = E // NT
K = 80
NCHUNK = EPT // K
NP = 10240
RPT = NP // NS
RB = 128
NB = RPT // RB


def _mm_body(h_ref, w_ref, o_ref):
    o_ref[...] = jnp.dot(h_ref[...], w_ref[...],
                         precision=lax.Precision.HIGHEST,
                         preferred_element_type=jnp.float32)


def _matmul(h, w):
    n, d = h.shape
    m = w.shape[1]
    bn = 512
    return pl.pallas_call(
        _mm_body,
        grid=(pl.cdiv(n, bn),),
        in_specs=[pl.BlockSpec((bn, d), lambda i: (i, 0)),
                  pl.BlockSpec((d, m), lambda i: (0, 0))],
        out_specs=pl.BlockSpec((bn, m), lambda i: (i, 0)),
        out_shape=jax.ShapeDtypeStruct((n, m), jnp.float32),
    )(h, w)


def _combine_body(relu, h_ref, w_ref, b_ref, a0_ref, a1_ref, dp_ref, o_ref):
    deg = jnp.sum(dp_ref[...], axis=0)[:, None]
    inv = 1.0 / jnp.maximum(deg, 1.0)
    o = (jnp.dot(h_ref[...], w_ref[...],
                 precision=lax.Precision.HIGHEST,
                 preferred_element_type=jnp.float32)
         + b_ref[...] + (a0_ref[...] + a1_ref[...]) * inv)
    if relu:
        o = jnp.maximum(o, 0.0)
    o_ref[...] = o


def _combine(h, w, b, a0, a1, dp, relu):
    n, d = h.shape
    m = w.shape[1]
    bn = 512
    return pl.pallas_call(
        functools.partial(_combine_body, relu),
        grid=(pl.cdiv(n, bn),),
        in_specs=[pl.BlockSpec((bn, d), lambda i: (i, 0)),
                  pl.BlockSpec((d, m), lambda i: (0, 0)),
                  pl.BlockSpec((1, m), lambda i: (0, 0)),
                  pl.BlockSpec((bn, m), lambda i: (i, 0)),
                  pl.BlockSpec((bn, m), lambda i: (i, 0)),
                  pl.BlockSpec((NT, bn), lambda i: (0, i))],
        out_specs=pl.BlockSpec((bn, m), lambda i: (i, 0)),
        out_shape=jax.ShapeDtypeStruct((n, m), jnp.float32),
    )(h, w, b, a0, a1, dp)


def _make_sc_agg(w, with_deg):
    mesh = plsc.VectorSubcoreMesh(core_axis_name="c", subcore_axis_name="s")
    out_type = [jax.ShapeDtypeStruct((NC, NP, w), jnp.float32)]
    scratch = [
        pltpu.VMEM_SHARED((NP, w), jnp.float32),
        pltpu.VMEM((2, K), jnp.int32),
        pltpu.VMEM((K, w), jnp.float32),
        pltpu.VMEM((RB, w), jnp.float32),
        pltpu.VMEM((RB,), jnp.int32),
        pltpu.SemaphoreType.DMA,
        pltpu.SemaphoreType.DMA,
    ]
    if with_deg:
        out_type.append(jax.ShapeDtypeStruct((NT, NP), jnp.float32))
        scratch.append(pltpu.VMEM((NP,), jnp.float32))

    @functools.partial(
        pl.kernel, out_type=out_type, mesh=mesh, scratch_types=scratch,
        compiler_params=pltpu.CompilerParams(needs_layout_passes=False))
    def sc_agg(*refs):
        if with_deg:
            (t_hbm, sd_hbm, z_hbm, zdeg_hbm,
             out_acc, out_deg,
             acc_sh, sd_v, rows_v, bounce_v, ix_v, sem, sem2, deg_v) = refs
        else:
            (t_hbm, sd_hbm, z_hbm,
             out_acc,
             acc_sh, sd_v, rows_v, bounce_v, ix_v, sem, sem2) = refs
        cid = lax.axis_index("c")
        sid = lax.axis_index("s")
        wid = sid * NC + cid
        row0 = sid * RPT
        iota = lax.iota(jnp.int32, 16)
        pltpu.sync_copy(z_hbm, bounce_v)
        if with_deg:
            pltpu.sync_copy(zdeg_hbm, deg_v)

        def fill_ix(r):
            for ii in range(RB // 16):
                ix_v[pl.ds(ii * 16, 16)] = iota + (r + ii * 16)

        def zero_chunk(i, carry):
            r = row0 + i * RB
            fill_ix(r)
            pltpu.sync_copy(bounce_v, acc_sh.at[ix_v])
            return carry

        lax.fori_loop(0, NB, zero_chunk, 0)
        plsc.subcore_barrier()
        cbase = wid * NCHUNK
        ones_lane = jnp.ones((16,), jnp.float32)

        def chunk(j, carry):
            pltpu.sync_copy(sd_hbm.at[cbase + j], sd_v)
            hA = pltpu.async_copy(t_hbm.at[sd_v.at[0, pl.ds(0, K // 2)]],
                                  rows_v.at[pl.ds(0, K // 2)], sem)
            hB = pltpu.async_copy(t_hbm.at[sd_v.at[0, pl.ds(K // 2, K // 2)]],
                                  rows_v.at[pl.ds(K // 2, K // 2)], sem2)
            hA.wait()
            hB.wait()
            pltpu.sync_copy(rows_v, acc_sh.at[sd_v.at[1]], add=True)
            if with_deg:
                for ii in range(K // 16):
                    idx = sd_v[1, pl.ds(ii * 16, 16)]
                    plsc.addupdate_scatter(deg_v, [idx], ones_lane)
            return carry

        lax.fori_loop(0, NCHUNK, chunk, 0)
        plsc.subcore_barrier()

        def read_chunk(i, carry):
            r = row0 + i * RB
            fill_ix(r)
            pltpu.async_copy(acc_sh.at[ix_v], bounce_v, sem).wait()
            pltpu.sync_copy(bounce_v,
                            out_acc.at[cid, pl.ds(pl.multiple_of(r, 8), RB)])
            return carry

        lax.fori_loop(0, NB, read_chunk, 0)
        if with_deg:
            pltpu.sync_copy(deg_v, out_deg.at[wid])

    return sc_agg


_sc_agg_deg = _make_sc_agg(H, True)
_sc_agg_h = _make_sc_agg(H, False)


def kernel(x, edge_index, W_self0, W_neigh0, b0, W_self1, W_neigh1, b1,
           W_self2, W_neigh2, b2):
    sd = jnp.stack([edge_index[0].reshape(NT * NCHUNK, K),
                    edge_index[1].reshape(NT * NCHUNK, K)], axis=1)
    zH = jnp.zeros((RB, H), jnp.float32)
    zC = jnp.zeros((RB, CP), jnp.float32)
    zdeg = jnp.zeros((NP,), jnp.float32)

    t0 = _matmul(x, W_neigh0)
    acc0, degp = _sc_agg_deg(t0, sd, zH, zdeg)
    dp = degp[:, :N]
    h1 = _combine(x, W_self0, b0.reshape(1, H), acc0[0, :N], acc0[1, :N],
                  dp, relu=True)

    t1 = _matmul(h1, W_neigh1)
    acc1 = _sc_agg_h(t1, sd, zH)[0]
    h2 = _combine(h1, W_self1, b1.reshape(1, H), acc1[0, :N], acc1[1, :N],
                  dp, relu=True)

    Wn2 = jnp.pad(W_neigh2, ((0, 0), (0, CP - C)))
    Ws2 = jnp.pad(W_self2, ((0, 0), (0, CP - C)))
    b2p = jnp.pad(b2, (0, CP - C)).reshape(1, CP)
    t2 = _matmul(h2, Wn2)
    acc2 = _sc_agg_h(t2, sd, zC)[0]
    out = _combine(h2, Ws2, b2p, acc2[0, :N], acc2[1, :N], dp, relu=False)
    return out[:, :C]
